# Initial kernel scaffold; baseline (speedup 1.0000x reference)
import jax
import jax.numpy as jnp
from jax.experimental import pallas as pl


def kernel(x, params):
    raise NotImplementedError("write your pallas kernel here")


# speedup vs baseline: 10.7058x; 10.7058x over previous
import functools

import jax
import jax.numpy as jnp
from jax.experimental import pallas as pl

_BN_EPS = 1e-5


def _fps_body(xyzT_ref, out_ref, *, M):
    x0 = xyzT_ref[:, 0, :]
    x1 = xyzT_ref[:, 1, :]
    x2 = xyzT_ref[:, 2, :]
    Bq, Nq = x0.shape
    iota_n = jax.lax.broadcasted_iota(jnp.int32, (Bq, Nq), 1)
    iota_m = jax.lax.broadcasted_iota(jnp.int32, (Bq, M), 1)

    def step(i, carry):
        dists, far, acc = carry
        acc = acc + (far - acc) * (iota_m == i).astype(jnp.int32)
        mask = iota_n == far
        c0 = jnp.sum(jnp.where(mask, x0, 0.0), axis=1, keepdims=True)
        c1 = jnp.sum(jnp.where(mask, x1, 0.0), axis=1, keepdims=True)
        c2 = jnp.sum(jnp.where(mask, x2, 0.0), axis=1, keepdims=True)
        d = (x0 - c0) ** 2 + (x1 - c1) ** 2 + (x2 - c2) ** 2
        dists = jnp.minimum(dists, d)
        maxv = jnp.max(dists, axis=1, keepdims=True)
        far = jnp.min(jnp.where(dists == maxv, iota_n, Nq), axis=1,
                      keepdims=True)
        return dists, far, acc

    dists0 = x0 * 0.0 + 1e10
    far0 = (x0[:, :1] * 0.0).astype(jnp.int32)
    acc0 = iota_m * 0
    _, _, acc = jax.lax.fori_loop(0, M, step, (dists0, far0, acc0))
    out_ref[...] = acc


def _fps(xyzT, M):
    Bq = xyzT.shape[0]
    return pl.pallas_call(
        functools.partial(_fps_body, M=M),
        out_shape=jax.ShapeDtypeStruct((Bq, M), jnp.int32),
    )(xyzT)


def _bq_body(idx_ref, tab_ref, new_xyz_ref, *g_refs, BM, radii, Ks):
    _, Nq, C = tab_ref.shape
    iota_mn = jax.lax.broadcasted_iota(jnp.int32, (BM, Nq), 1)
    tab = tab_ref[0]
    xyzb = tab[:, :3]
    fidx = idx_ref[0, 0, 0]
    oh_m = (fidx[:, None] == iota_mn).astype(jnp.float32)
    P = jax.lax.dot_general(
        oh_m, xyzb, (((1,), (0,)), ((), ())),
        preferred_element_type=jnp.float32,
            precision=jax.lax.Precision.HIGHEST)
    new_xyz_ref[0] = P
    pn = jnp.sum(P * P, axis=1, keepdims=True)
    qn = jnp.sum(xyzb * xyzb, axis=1)[None, :]
    G = jax.lax.dot_general(
        P, xyzb, (((1,), (1,)), ((), ())),
        preferred_element_type=jnp.float32)
    d = (-2.0 * G + pn) + qn
    sub = jnp.concatenate(
        [P, jnp.zeros((BM, C - 3), jnp.float32)], axis=1)
    for j, (r, K) in enumerate(zip(radii, Ks)):
        key = jnp.where(d > r * r, Nq, iota_mn)
        m_first = None
        for k in range(K):
            m = jnp.min(key, axis=1, keepdims=True)
            if k == 0:
                m_first = jnp.minimum(m, Nq - 1)
                m_eff = m_first
            else:
                m_eff = jnp.where(m == Nq, m_first, m)
            oh = (iota_mn == m_eff).astype(jnp.float32)
            row = jax.lax.dot_general(
                oh, tab, (((1,), (0,)), ((), ())),
                preferred_element_type=jnp.float32,
            precision=jax.lax.Precision.HIGHEST)
            g_refs[j][0, k] = row - sub
            key = jnp.where(key == m, Nq, key)


def _ballquery(tab, fps_idx, radii, Ks):
    Bq, Nq, C = tab.shape
    M = fps_idx.shape[1]
    BM = min(M, 128)
    outs = [jax.ShapeDtypeStruct((Bq, M, 3), jnp.float32)]
    outs += [jax.ShapeDtypeStruct((Bq, K, M, C), jnp.float32) for K in Ks]
    out_specs = [pl.BlockSpec((1, BM, 3), lambda b, m: (b, m, 0))]
    out_specs += [
        pl.BlockSpec((1, K, BM, C), lambda b, m: (b, 0, m, 0)) for K in Ks]
    idx4 = fps_idx.reshape(Bq, M // BM, 1, BM)
    res = pl.pallas_call(
        functools.partial(_bq_body, BM=BM, radii=tuple(radii), Ks=tuple(Ks)),
        grid=(Bq, M // BM),
        in_specs=[
            pl.BlockSpec((1, 1, 1, BM), lambda b, m: (b, m, 0, 0)),
            pl.BlockSpec((1, Nq, C), lambda b, m: (b, 0, 0)),
        ],
        out_shape=tuple(outs),
        out_specs=tuple(out_specs),
    )(idx4, tab)
    return res[0], res[1:]


_CHUNK = 2048


def _layer_chunked(in_ref, out_ref, w_ref, b_ref, ga_ref, be_ref, R):
    ch = min(R, _CHUNK)
    n = R // ch
    w = w_ref[...]
    b = b_ref[...][None, :]
    s = None
    for i in range(n):
        h = in_ref[pl.ds(i * ch, ch), :]
        y = jax.lax.dot_general(
            h, w, (((1,), (1,)), ((), ())),
            preferred_element_type=jnp.float32) + b
        out_ref[pl.ds(i * ch, ch), :] = y
        ps = jnp.sum(y, axis=0, keepdims=True)
        s = ps if s is None else s + ps
    mean = s * (1.0 / R)
    v = None
    for i in range(n):
        y = out_ref[pl.ds(i * ch, ch), :]
        dy = y - mean
        pv = jnp.sum(dy * dy, axis=0, keepdims=True)
        v = pv if v is None else v + pv
    var = v * (1.0 / R)
    denom = jnp.sqrt(var + _BN_EPS)
    ga = ga_ref[...][None, :]
    be = be_ref[...][None, :]
    for i in range(n):
        y = out_ref[pl.ds(i * ch, ch), :]
        out_ref[pl.ds(i * ch, ch), :] = \
            jnp.maximum((y - mean) / denom * ga + be, 0.0)


def _layer_t(src_ref, C_in, dst_ref, w_ref, b_ref, ga_ref, be_ref, R):
    ch = min(R, 4096)
    n = R // ch
    w = w_ref[...]
    C_out = w.shape[0]
    bcol = b_ref[...]
    s = None
    for i in range(n):
        xc = src_ref[0:C_in, i * ch:(i + 1) * ch]
        y = jax.lax.dot_general(
            w, xc, (((1,), (0,)), ((), ())),
            preferred_element_type=jnp.float32) + bcol
        dst_ref[0:C_out, i * ch:(i + 1) * ch] = y
        ps = jnp.sum(y, axis=1, keepdims=True)
        s = ps if s is None else s + ps
    mean = s * (1.0 / R)
    v = None
    for i in range(n):
        y = dst_ref[0:C_out, i * ch:(i + 1) * ch]
        dy = y - mean
        pv = jnp.sum(dy * dy, axis=1, keepdims=True)
        v = pv if v is None else v + pv
    var = v * (1.0 / R)
    denom = jnp.sqrt(var + _BN_EPS)
    ga = ga_ref[...]
    be = be_ref[...]
    for i in range(n):
        y = dst_ref[0:C_out, i * ch:(i + 1) * ch]
        dst_ref[0:C_out, i * ch:(i + 1) * ch] = \
            jnp.maximum((y - mean) / denom * ga + be, 0.0)


def _mlp_body(xT_ref, *refs, B, K, M, n_layers, widths):
    layer_refs = [tuple(refs[4 * i:4 * i + 4]) for i in range(n_layers)]
    out_ref = refs[4 * n_layers]
    bufs = refs[4 * n_layers + 1:]
    C_in = xT_ref.shape[0]
    R = xT_ref.shape[1]
    src = xT_ref
    for li, lr in enumerate(layer_refs):
        dst = bufs[li % 2]
        _layer_t(src, C_in, dst, *lr, R)
        src = dst
        C_in = widths[li]
    C3 = widths[-1]
    for b in range(B):
        acc = src[0:C3, (b * K) * M:(b * K + 1) * M]
        for k in range(1, K):
            lo = (b * K + k) * M
            acc = jnp.maximum(acc, src[0:C3, lo:lo + M])
        out_ref[b] = acc


def _mlp(gT, layers, B, K, M):
    from jax.experimental.pallas import tpu as pltpu
    R = B * K * M
    widths = tuple(l['W'].shape[0] for l in layers)
    args = [gT]
    for l in layers:
        args += [l['W'], l['b'][:, None], l['gamma'][:, None],
                 l['beta'][:, None]]
    Cmax = max(widths)
    scratch = [pltpu.VMEM((Cmax, R), jnp.float32) for _ in range(2)]
    return pl.pallas_call(
        functools.partial(_mlp_body, B=B, K=K, M=M, n_layers=len(layers),
                          widths=widths),
        out_shape=jax.ShapeDtypeStruct((B, widths[-1], M), jnp.float32),
        scratch_shapes=scratch,
    )(*args)


def _sa3_body(x_ref, *refs, B, M, n_layers):
    layer_refs = [tuple(refs[4 * i:4 * i + 4]) for i in range(n_layers)]
    fc_ref = refs[4 * n_layers]
    out_ref = refs[4 * n_layers + 1]
    scratch = refs[4 * n_layers + 2:]
    R = x_ref.shape[0]
    src = x_ref
    for li, (w_ref, b_ref, ga_ref, be_ref) in enumerate(layer_refs):
        _layer_chunked(src, scratch[li], w_ref, b_ref, ga_ref, be_ref, R)
        src = scratch[li]
    hN = scratch[n_layers - 1]
    C3 = hN.shape[1]
    net = None
    for b in range(B):
        acc = jnp.max(hN[pl.ds(b * M, M), :], axis=0, keepdims=True)
        net = acc if net is None else jnp.concatenate([net, acc], axis=0)
    out_ref[...] = jax.lax.dot_general(
        net, fc_ref[...], (((1,), (1,)), ((), ())),
        preferred_element_type=jnp.float32)


def _sa3_fc(grouped, layers, fc1, B, M):
    from jax.experimental.pallas import tpu as pltpu
    R = grouped.shape[0]
    args = [grouped]
    for l in layers:
        args += [l['W'], l['b'], l['gamma'], l['beta']]
    args.append(fc1)
    scratch = [pltpu.VMEM((R, l['W'].shape[0]), jnp.float32) for l in layers]
    return pl.pallas_call(
        functools.partial(_sa3_body, B=B, M=M, n_layers=len(layers)),
        out_shape=jax.ShapeDtypeStruct((B, fc1.shape[0]), jnp.float32),
        scratch_shapes=scratch,
    )(*args)


def _sa_msg(tab, M, radii, Ks, branch_params):
    B = tab.shape[0]
    xyzT = jnp.transpose(tab[..., :3], (0, 2, 1))
    fps_idx = _fps(xyzT, M)
    new_xyz, gs = _ballquery(tab, fps_idx, radii, Ks)
    outs = []
    for g, K, layers in zip(gs, Ks, branch_params):
        C = g.shape[-1]
        gT = jnp.transpose(g.reshape(B * K * M, C))
        outs.append(_mlp(gT, layers, B, K, M))
    new_pointsT = jnp.concatenate(outs, axis=1)
    return new_xyz, jnp.transpose(new_pointsT, (0, 2, 1))


def kernel(x, params):
    B = x.shape[0]
    new_xyz, new_points = _sa_msg(
        x, 512, (0.1, 0.2, 0.4), (2, 4, 8), params['sa1'])
    tab2 = jnp.concatenate([new_xyz, new_points], axis=-1)
    new_xyz2, new_points2 = _sa_msg(
        tab2, 128, (0.2, 0.4, 0.8), (2, 4, 8), params['sa2'])
    grouped = jnp.concatenate([new_xyz2, new_points2], axis=-1)
    grouped = grouped.reshape(B * 128, grouped.shape[-1])
    return _sa3_fc(grouped, params['sa3'], params['fc1'], B, 128)
